# Initial kernel scaffold; baseline (speedup 1.0000x reference)
import jax
import jax.numpy as jnp
from jax.experimental import pallas as pl


def kernel(x, W, b, mask):
    raise NotImplementedError("write your pallas kernel here")


# speedup vs baseline: 1.5342x; 1.5342x over previous
import functools

import jax
import jax.numpy as jnp
from jax.experimental import pallas as pl

V = 55


def _var_roll(y, r):
    for bit in range(6):
        amt = 1 << bit
        rolled = jnp.roll(y, -amt, axis=2)
        cond = ((r >> bit) & 1) == 1
        y = jnp.where(cond, rolled, y)
    return y


def _body(x_ref, w_ref, b_ref, m_ref, o_ref):
    xs = x_ref[0]
    c_dim, t_dim, v_dim = xs.shape

    c_idx = jax.lax.broadcasted_iota(jnp.int32, (c_dim, 1, 1), 0)
    y = _var_roll(xs, c_idx % V)

    zero_col = jnp.zeros_like(y[:, :, :1])
    left = jnp.concatenate([zero_col, y[:, :, :-1]], axis=2)
    right = jnp.concatenate([y[:, :, 1:], zero_col], axis=2)
    agg = y + left + right
    i_idx = jax.lax.broadcasted_iota(jnp.int32, (1, 1, v_dim), 2)
    recip_deg = jnp.where((i_idx == 0) | (i_idx == v_dim - 1),
                          jnp.float32(0.5), jnp.float32(1.0 / 3.0))
    agg = agg * recip_deg

    m2 = jnp.tanh(m_ref[...]) + 1.0
    agg = agg * m2[:, None, :]

    hm2 = agg.reshape(c_dim, t_dim * v_dim)
    h = jax.lax.dot_general(w_ref[...], hm2, (((0,), (0,)), ((), ())),
                            preferred_element_type=jnp.float32)
    h = h + b_ref[...]
    h3 = h.reshape(h.shape[0], t_dim, v_dim)

    d_idx = jax.lax.broadcasted_iota(jnp.int32, (h.shape[0], 1, 1), 0)
    out = _var_roll(h3, d_idx % V)
    o_ref[0] = jnp.maximum(out, 0.0)


@jax.jit
def kernel(x, W, b, mask):
    n, c, t, v = x.shape
    d = W.shape[1]
    m_t = jnp.transpose(mask[0], (1, 0))
    b2 = b.reshape(d, 1)

    out = pl.pallas_call(
        _body,
        grid=(n,),
        in_specs=[
            pl.BlockSpec((1, c, t, v), lambda i: (i, 0, 0, 0)),
            pl.BlockSpec((c, d), lambda i: (0, 0)),
            pl.BlockSpec((d, 1), lambda i: (0, 0)),
            pl.BlockSpec((c, v), lambda i: (0, 0)),
        ],
        out_specs=pl.BlockSpec((1, d, t, v), lambda i: (i, 0, 0, 0)),
        out_shape=jax.ShapeDtypeStruct((n, d, t, v), jnp.float32),
    )(x, W, b2, m_t)
    return out


# speedup vs baseline: 2.3278x; 1.5173x over previous
import functools

import jax
import jax.numpy as jnp
from jax.experimental import pallas as pl

V = 55


def _var_roll(y):
    c_dim = y.shape[0]
    parts = []
    for c in range(c_dim):
        amt = c % V
        sl = y[c]
        if amt:
            sl = jnp.concatenate([sl[:, amt:], sl[:, :amt]], axis=1)
        parts.append(sl[None])
    return jnp.concatenate(parts, axis=0)


def _body(x_ref, w_ref, b_ref, m_ref, o_ref):
    xs = x_ref[0]
    c_dim, t_dim, v_dim = xs.shape

    y = _var_roll(xs)

    zero_col = jnp.zeros_like(y[:, :, :1])
    left = jnp.concatenate([zero_col, y[:, :, :-1]], axis=2)
    right = jnp.concatenate([y[:, :, 1:], zero_col], axis=2)
    agg = y + left + right

    i_idx = jax.lax.broadcasted_iota(jnp.int32, (1, v_dim), 1)
    recip_deg = jnp.where((i_idx == 0) | (i_idx == v_dim - 1),
                          jnp.float32(0.5), jnp.float32(1.0 / 3.0))
    scale = (jnp.tanh(m_ref[...]) + 1.0) * recip_deg
    agg = agg * scale[:, None, :]

    hm2 = agg.reshape(c_dim, t_dim * v_dim)
    h = jax.lax.dot_general(w_ref[...], hm2, (((0,), (0,)), ((), ())),
                            preferred_element_type=jnp.float32)
    h = jnp.maximum(h + b_ref[...], 0.0)
    h3 = h.reshape(h.shape[0], t_dim, v_dim)

    o_ref[0] = _var_roll(h3)


@jax.jit
def kernel(x, W, b, mask):
    n, c, t, v = x.shape
    d = W.shape[1]
    m_t = jnp.transpose(mask[0], (1, 0))
    b2 = b.reshape(d, 1)

    out = pl.pallas_call(
        _body,
        grid=(n,),
        in_specs=[
            pl.BlockSpec((1, c, t, v), lambda i: (i, 0, 0, 0)),
            pl.BlockSpec((c, d), lambda i: (0, 0)),
            pl.BlockSpec((d, 1), lambda i: (0, 0)),
            pl.BlockSpec((c, v), lambda i: (0, 0)),
        ],
        out_specs=pl.BlockSpec((1, d, t, v), lambda i: (i, 0, 0, 0)),
        out_shape=jax.ShapeDtypeStruct((n, d, t, v), jnp.float32),
    )(x, W, b2, m_t)
    return out


# speedup vs baseline: 3.8769x; 1.6655x over previous
import functools

import jax
import jax.numpy as jnp
import numpy as np
from jax.experimental import pallas as pl

V = 55


def _shift_stencil_mats(c_dim):
    deg = np.full(V, 3.0, np.float32)
    deg[0] = deg[-1] = 2.0
    k = np.arange(V)
    A = ((np.abs(k[:, None] - k[None, :]) <= 1).astype(np.float32)
         / deg[None, :])
    B = np.zeros((c_dim, V, V), np.float32)
    for c in range(c_dim):
        r = c % V
        P = np.zeros((V, V), np.float32)
        P[(k + r) % V, k] = 1.0
        B[c] = P @ A
    return B


def _out_shift_mats(d_dim):
    k = np.arange(V)
    P = np.zeros((d_dim, V, V), np.float32)
    for d in range(d_dim):
        P[d, (k + d % V) % V, k] = 1.0
    return P


def _body(x_ref, w_ref, b_ref, m_ref, bmat_ref, pout_ref, o_ref):
    xs = x_ref[0]
    c_dim, t_dim, v_dim = xs.shape

    scale = jnp.tanh(m_ref[...]) + 1.0
    bm = bmat_ref[...] * scale[:, None, :]

    agg = jax.lax.dot_general(xs, bm, (((2,), (1,)), ((0,), (0,))),
                              preferred_element_type=jnp.float32)

    hm2 = agg.reshape(c_dim, t_dim * v_dim)
    h = jax.lax.dot_general(w_ref[...], hm2, (((0,), (0,)), ((), ())),
                            preferred_element_type=jnp.float32)
    h = jnp.maximum(h + b_ref[...], 0.0)
    h3 = h.reshape(h.shape[0], t_dim, v_dim)

    o_ref[0] = jax.lax.dot_general(h3, pout_ref[...],
                                   (((2,), (1,)), ((0,), (0,))),
                                   preferred_element_type=jnp.float32)


@jax.jit
def kernel(x, W, b, mask):
    n, c, t, v = x.shape
    d = W.shape[1]
    m_t = jnp.transpose(mask[0], (1, 0))
    b2 = b.reshape(d, 1)
    bmat = jnp.asarray(_shift_stencil_mats(c))
    pout = jnp.asarray(_out_shift_mats(d))

    out = pl.pallas_call(
        _body,
        grid=(n,),
        in_specs=[
            pl.BlockSpec((1, c, t, v), lambda i: (i, 0, 0, 0)),
            pl.BlockSpec((c, d), lambda i: (0, 0)),
            pl.BlockSpec((d, 1), lambda i: (0, 0)),
            pl.BlockSpec((c, v), lambda i: (0, 0)),
            pl.BlockSpec((c, v, v), lambda i: (0, 0, 0)),
            pl.BlockSpec((d, v, v), lambda i: (0, 0, 0)),
        ],
        out_specs=pl.BlockSpec((1, d, t, v), lambda i: (i, 0, 0, 0)),
        out_shape=jax.ShapeDtypeStruct((n, d, t, v), jnp.float32),
    )(x, W, b2, m_t, bmat, pout)
    return out
